# baseline (speedup 1.0000x reference)
import jax
import jax.numpy as jnp
from jax.experimental import pallas as pl
from jax.experimental.pallas import tpu as pltpu

D_MODEL = 4096
NUM_EXPERTS = 64
TOP_K = 8
BT = 2048
BD = 2048


def _gating_kernel(x_ref, w_ref, gates_ref, idx_ref, acc_ref):
    j = pl.program_id(1)
    nd = pl.num_programs(1)
    part = jax.lax.dot_general(
        w_ref[...], x_ref[...], (((1,), (1,)), ((), ())),
        preferred_element_type=jnp.float32)

    @pl.when(j == 0)
    def _init():
        acc_ref[...] = part

    @pl.when(j > 0)
    def _accum():
        acc_ref[...] += part

    @pl.when(j == nd - 1)
    def _finish():
        logits = acc_ref[...]
        iota = jax.lax.broadcasted_iota(jnp.int32, logits.shape, 0)
        work = logits
        vals, idxs = [], []
        for _ in range(TOP_K):
            m = jnp.max(work, axis=0, keepdims=True)
            hit = jnp.min(jnp.where(work == m, iota, NUM_EXPERTS),
                          axis=0, keepdims=True)
            vals.append(m)
            idxs.append(hit)
            work = jnp.where(iota == hit, -jnp.inf, work)
        v = jnp.concatenate(vals, axis=0)
        ix = jnp.concatenate(idxs, axis=0)
        e = jnp.exp(v - v[:1])
        gates_ref[...] = e / jnp.sum(e, axis=0, keepdims=True)
        idx_ref[...] = ix


def kernel(x, W_gate, W_noise):
    B, N, D = x.shape
    T = B * N
    xf = x.reshape(T, D)
    gates_t, idx_t = pl.pallas_call(
        _gating_kernel,
        grid=(T // BT, D // BD),
        in_specs=[
            pl.BlockSpec((BT, BD), lambda i, j: (i, j)),
            pl.BlockSpec((NUM_EXPERTS, BD), lambda i, j: (0, j)),
        ],
        out_specs=[
            pl.BlockSpec((TOP_K, BT), lambda i, j: (0, i)),
            pl.BlockSpec((TOP_K, BT), lambda i, j: (0, i)),
        ],
        out_shape=[
            jax.ShapeDtypeStruct((TOP_K, T), jnp.float32),
            jax.ShapeDtypeStruct((TOP_K, T), jnp.int32),
        ],
        scratch_shapes=[pltpu.VMEM((NUM_EXPERTS, BT), jnp.float32)],
        compiler_params=pltpu.CompilerParams(
            dimension_semantics=("parallel", "arbitrary")),
    )(xf, W_gate)
    gates = gates_t.T.reshape(B, N, TOP_K)
    idx = idx_t.T.reshape(B, N, TOP_K)
    return gates, idx
